# baseline (device time: 15768 ns/iter reference)
import jax
import jax.numpy as jnp
from jax import lax
from jax.experimental import pallas as pl
from jax.experimental.pallas import tpu as pltpu

N_DEV = 4


def kernel(x, w_mat):
    m_per, k = x.shape
    _, n = w_mat.shape
    n_per = n // N_DEV

    def body(x_ref, w_hbm, out_ref, w_buf, comm_ref, load_sems,
             send_sems, recv_sems):
        my = lax.axis_index("i")
        targets = [(my + 2) % N_DEV, (my + 1) % N_DEV,
                   (my + 3) % N_DEV, my]

        def start_load(idx, slot):
            cp = pltpu.make_async_copy(
                w_hbm.at[:, pl.ds(targets[idx] * n_per, n_per)],
                w_buf.at[slot],
                load_sems.at[slot],
            )
            cp.start()
            return cp

        loads = [start_load(0, 0)]

        barrier_sem = pltpu.get_barrier_semaphore()
        for d in range(1, N_DEV):
            pl.semaphore_signal(
                barrier_sem, inc=1,
                device_id=((my + d) % N_DEV,),
                device_id_type=pl.DeviceIdType.MESH,
            )
        pl.semaphore_wait(barrier_sem, N_DEV - 1)

        rdmas = []
        for i in range(N_DEV):
            loads[i].wait()
            if i + 1 < N_DEV:
                loads.append(start_load(i + 1, (i + 1) % 2))
            yc = jnp.dot(
                x_ref[:, :], w_buf[i % 2, :, :],
                preferred_element_type=jnp.float32,
            )
            yc = yc * jax.nn.sigmoid(yc)
            if i < N_DEV - 1:
                comm_ref[i, :, :] = yc
                rdma = pltpu.make_async_remote_copy(
                    src_ref=comm_ref.at[i],
                    dst_ref=out_ref.at[pl.ds(my * m_per, m_per), :],
                    send_sem=send_sems.at[i],
                    recv_sem=recv_sems.at[i],
                    device_id=(targets[i],),
                    device_id_type=pl.DeviceIdType.MESH,
                )
                rdma.start()
                rdmas.append(rdma)
            else:
                out_ref[pl.ds(my * m_per, m_per), :] = yc

        for rdma in rdmas:
            rdma.wait()

    out_shape = jax.ShapeDtypeStruct((N_DEV * m_per, n_per), jnp.float32)
    return pl.pallas_call(
        body,
        out_shape=out_shape,
        in_specs=[
            pl.BlockSpec(memory_space=pltpu.VMEM),
            pl.BlockSpec(memory_space=pl.ANY),
        ],
        out_specs=pl.BlockSpec(memory_space=pltpu.VMEM),
        scratch_shapes=[
            pltpu.VMEM((2, k, n_per), jnp.float32),
            pltpu.VMEM((N_DEV - 1, m_per, n_per), jnp.float32),
            pltpu.SemaphoreType.DMA((2,)),
            pltpu.SemaphoreType.DMA((N_DEV - 1,)),
            pltpu.SemaphoreType.DMA((N_DEV - 1,)),
        ],
        compiler_params=pltpu.CompilerParams(collective_id=0),
    )(x, w_mat)


# device time: 13138 ns/iter; 1.2002x vs baseline; 1.2002x over previous
import jax
import jax.numpy as jnp
from jax import lax
from jax.experimental import pallas as pl
from jax.experimental.pallas import tpu as pltpu

N_DEV = 4
DISTS = (2, 1, 3)


def kernel(x, w_mat):
    m_per, k = x.shape
    _, n = w_mat.shape
    n_per = n // N_DEV

    def body(x_ref, w_ref, out_ref, y_ref, send_buf, recv_buf,
             send_sems, recv_sems):
        my = lax.axis_index("i")

        barrier_sem = pltpu.get_barrier_semaphore()
        for d in range(1, N_DEV):
            pl.semaphore_signal(
                barrier_sem, inc=1,
                device_id=((my + d) % N_DEV,),
                device_id_type=pl.DeviceIdType.MESH,
            )
        pl.semaphore_wait(barrier_sem, N_DEV - 1)

        y = jnp.dot(x_ref[:, :], w_ref[:, :], preferred_element_type=jnp.float32)
        y_ref[:, :] = y * jax.nn.sigmoid(y)

        rdmas = []
        for i, d in enumerate(DISTS):
            tgt = (my + d) % N_DEV
            send_buf[i, :, :] = y_ref[:, pl.ds(tgt * n_per, n_per)].astype(
                jnp.bfloat16
            )
            rdma = pltpu.make_async_remote_copy(
                src_ref=send_buf.at[i],
                dst_ref=recv_buf.at[i],
                send_sem=send_sems.at[i],
                recv_sem=recv_sems.at[i],
                device_id=(tgt,),
                device_id_type=pl.DeviceIdType.MESH,
            )
            rdma.start()
            rdmas.append(rdma)

        out_ref[pl.ds(my * m_per, m_per), :] = y_ref[:, pl.ds(my * n_per, n_per)]

        for i in (1, 2, 0):
            rdmas[i].wait_recv()
            src = (my - DISTS[i]) % N_DEV
            out_ref[pl.ds(src * m_per, m_per), :] = recv_buf[i, :, :].astype(
                jnp.float32
            )
        for rdma in rdmas:
            rdma.wait_send()

    out_shape = jax.ShapeDtypeStruct((N_DEV * m_per, n_per), jnp.float32)
    return pl.pallas_call(
        body,
        out_shape=out_shape,
        in_specs=[
            pl.BlockSpec(memory_space=pltpu.VMEM),
            pl.BlockSpec(memory_space=pltpu.VMEM),
        ],
        out_specs=pl.BlockSpec(memory_space=pltpu.VMEM),
        scratch_shapes=[
            pltpu.VMEM((m_per, n), jnp.float32),
            pltpu.VMEM((N_DEV - 1, m_per, n_per), jnp.bfloat16),
            pltpu.VMEM((N_DEV - 1, m_per, n_per), jnp.bfloat16),
            pltpu.SemaphoreType.DMA((N_DEV - 1,)),
            pltpu.SemaphoreType.DMA((N_DEV - 1,)),
        ],
        compiler_params=pltpu.CompilerParams(collective_id=0),
    )(x, w_mat)


# device time: 12715 ns/iter; 1.2401x vs baseline; 1.0333x over previous
import jax
import jax.numpy as jnp
from jax import lax
from jax.experimental import pallas as pl
from jax.experimental.pallas import tpu as pltpu

N_DEV = 4
DISTS = (2, 1, 3)


def kernel(x, w_mat):
    m_per, k = x.shape
    _, n = w_mat.shape
    n_per = n // N_DEV

    def body(x_ref, w_ref, out_ref, send_buf, recv_buf,
             send_sems, recv_sems):
        my = lax.axis_index("i")

        barrier_sem = pltpu.get_barrier_semaphore()
        for d in range(1, N_DEV):
            pl.semaphore_signal(
                barrier_sem, inc=1,
                device_id=((my + d) % N_DEV,),
                device_id_type=pl.DeviceIdType.MESH,
            )
        pl.semaphore_wait(barrier_sem, N_DEV - 1)

        rdmas = []
        for i, d in enumerate(DISTS):
            tgt = (my + d) % N_DEV
            yc = jnp.dot(
                x_ref[:, :], w_ref[:, pl.ds(tgt * n_per, n_per)],
                preferred_element_type=jnp.float32,
            )
            send_buf[i, :, :] = (yc * jax.nn.sigmoid(yc)).astype(jnp.bfloat16)
            rdma = pltpu.make_async_remote_copy(
                src_ref=send_buf.at[i],
                dst_ref=recv_buf.at[i],
                send_sem=send_sems.at[i],
                recv_sem=recv_sems.at[i],
                device_id=(tgt,),
                device_id_type=pl.DeviceIdType.MESH,
            )
            rdma.start()
            rdmas.append(rdma)

        yc = jnp.dot(
            x_ref[:, :], w_ref[:, pl.ds(my * n_per, n_per)],
            preferred_element_type=jnp.float32,
        )
        out_ref[pl.ds(my * m_per, m_per), :] = yc * jax.nn.sigmoid(yc)

        for i in (1, 2, 0):
            rdmas[i].wait_recv()
            src = (my - DISTS[i]) % N_DEV
            out_ref[pl.ds(src * m_per, m_per), :] = recv_buf[i, :, :].astype(
                jnp.float32
            )
        for rdma in rdmas:
            rdma.wait_send()

    out_shape = jax.ShapeDtypeStruct((N_DEV * m_per, n_per), jnp.float32)
    return pl.pallas_call(
        body,
        out_shape=out_shape,
        in_specs=[
            pl.BlockSpec(memory_space=pltpu.VMEM),
            pl.BlockSpec(memory_space=pltpu.VMEM),
        ],
        out_specs=pl.BlockSpec(memory_space=pltpu.VMEM),
        scratch_shapes=[
            pltpu.VMEM((N_DEV - 1, m_per, n_per), jnp.bfloat16),
            pltpu.VMEM((N_DEV - 1, m_per, n_per), jnp.bfloat16),
            pltpu.SemaphoreType.DMA((N_DEV - 1,)),
            pltpu.SemaphoreType.DMA((N_DEV - 1,)),
        ],
        compiler_params=pltpu.CompilerParams(collective_id=0),
    )(x, w_mat)
